# initial kernel scaffold (unmeasured)
import jax
import jax.numpy as jnp
from jax import lax
from jax.experimental import pallas as pl
from jax.experimental.pallas import tpu as pltpu

N_DEV = 8


def kernel(x, router, W1, W2):
    t_per, d = x.shape
    e_per = W1.shape[0]
    f = W1.shape[2]
    t = t_per * N_DEV
    e_tot = e_per * N_DEV

    def body(x_ref, r_ref, w1_ref, w2_ref, o_ref,
             xf, rf, wts, acc, commx, commr, commrs,
             sx_send, sx_recv, sr_send, sr_recv, ss_send, ss_recv):
        e = pl.program_id(0)
        my = lax.axis_index("i")
        left = lax.rem(my - 1 + N_DEV, N_DEV)
        right = lax.rem(my + 1, N_DEV)

        @pl.when(e == 0)
        def _gather_and_route():
            barrier = pltpu.get_barrier_semaphore()
            for nbr in (left, right):
                pl.semaphore_signal(
                    barrier, inc=1, device_id=(nbr,),
                    device_id_type=pl.DeviceIdType.MESH,
                )
            pl.semaphore_wait(barrier, 2)

            xf[my] = x_ref[...]
            commx[0] = x_ref[...]
            for h in range(N_DEV - 1):
                s_slot = h % 2
                r_slot = (h + 1) % 2
                rdma = pltpu.make_async_remote_copy(
                    src_ref=commx.at[s_slot],
                    dst_ref=commx.at[r_slot],
                    send_sem=sx_send.at[s_slot],
                    recv_sem=sx_recv.at[r_slot],
                    device_id=(right,),
                    device_id_type=pl.DeviceIdType.MESH,
                )
                rdma.start()
                rdma.wait()
                origin = lax.rem(my - h - 1 + N_DEV, N_DEV)
                xf[origin] = commx[r_slot]

            rf[my] = r_ref[...]
            commr[0] = r_ref[...]
            for h in range(N_DEV - 1):
                s_slot = h % 2
                r_slot = (h + 1) % 2
                rdma = pltpu.make_async_remote_copy(
                    src_ref=commr.at[s_slot],
                    dst_ref=commr.at[r_slot],
                    send_sem=sr_send.at[s_slot],
                    recv_sem=sr_recv.at[r_slot],
                    device_id=(right,),
                    device_id_type=pl.DeviceIdType.MESH,
                )
                rdma.start()
                rdma.wait()
                origin = lax.rem(my - h - 1 + N_DEV, N_DEV)
                rf[origin] = commr[r_slot]

            xfull = xf[...].reshape(t, d)
            gates = jnp.concatenate(
                [
                    jnp.dot(xfull, rf[p], preferred_element_type=jnp.float32)
                    for p in range(N_DEV)
                ],
                axis=1,
            )

            m1 = jnp.max(gates, axis=1, keepdims=True)
            masked = jnp.where(gates >= m1, -jnp.inf, gates)
            m2 = jnp.max(masked, axis=1, keepdims=True)
            a = jnp.exp(m2 - m1)
            w_top = 1.0 / (1.0 + a)
            w_sec = a / (1.0 + a)
            wts[...] = jnp.where(gates >= m1, w_top, 0.0) + jnp.where(
                gates >= m2, w_sec, 0.0
            ) * jnp.where(gates >= m1, 0.0, 1.0)

        xfull = xf[...].reshape(t, d)
        h_act = jnp.maximum(
            jnp.dot(xfull, w1_ref[0], preferred_element_type=jnp.float32), 0.0
        )
        y = jnp.dot(h_act, w2_ref[0], preferred_element_type=jnp.float32)
        glob_e = my * e_per + e
        lane = lax.broadcasted_iota(jnp.int32, (t, e_tot), 1)
        col = jnp.sum(
            jnp.where(lane == glob_e, wts[...], 0.0), axis=1, keepdims=True
        )
        contrib = y * col

        @pl.when(e == 0)
        def _init_acc():
            acc[...] = contrib

        @pl.when(e > 0)
        def _add_acc():
            acc[...] = acc[...] + contrib

        @pl.when(e == e_per - 1)
        def _reduce_scatter():
            c0 = lax.rem(my - 1 + N_DEV, N_DEV)
            commrs[0] = acc[...].reshape(N_DEV, t_per, d)[c0]
            for s in range(N_DEV - 1):
                s_slot = s % 2
                r_slot = (s + 1) % 2
                rdma = pltpu.make_async_remote_copy(
                    src_ref=commrs.at[s_slot],
                    dst_ref=commrs.at[r_slot],
                    send_sem=ss_send.at[s_slot],
                    recv_sem=ss_recv.at[r_slot],
                    device_id=(right,),
                    device_id_type=pl.DeviceIdType.MESH,
                )
                rdma.start()
                rdma.wait()
                chunk = lax.rem(my - 2 - s + 2 * N_DEV, N_DEV)
                commrs[r_slot] = (
                    commrs[r_slot] + acc[...].reshape(N_DEV, t_per, d)[chunk]
                )
            o_ref[...] = commrs[(N_DEV - 1) % 2]

    grid = (e_per,)
    return pl.pallas_call(
        body,
        grid=grid,
        in_specs=[
            pl.BlockSpec((t_per, d), lambda e: (0, 0)),
            pl.BlockSpec((t, e_per), lambda e: (0, 0)),
            pl.BlockSpec((1, d, f), lambda e: (e, 0, 0)),
            pl.BlockSpec((1, f, d), lambda e: (e, 0, 0)),
        ],
        out_specs=pl.BlockSpec((t_per, d), lambda e: (0, 0)),
        out_shape=jax.ShapeDtypeStruct((t_per, d), jnp.float32),
        scratch_shapes=[
            pltpu.VMEM((N_DEV, t_per, d), jnp.float32),
            pltpu.VMEM((N_DEV, t, e_per), jnp.float32),
            pltpu.VMEM((t, e_tot), jnp.float32),
            pltpu.VMEM((t, d), jnp.float32),
            pltpu.VMEM((2, t_per, d), jnp.float32),
            pltpu.VMEM((2, t, e_per), jnp.float32),
            pltpu.VMEM((2, t_per, d), jnp.float32),
            pltpu.SemaphoreType.DMA((2,)),
            pltpu.SemaphoreType.DMA((2,)),
            pltpu.SemaphoreType.DMA((2,)),
            pltpu.SemaphoreType.DMA((2,)),
            pltpu.SemaphoreType.DMA((2,)),
            pltpu.SemaphoreType.DMA((2,)),
        ],
        compiler_params=pltpu.CompilerParams(
            dimension_semantics=("arbitrary",),
            collective_id=0,
        ),
    )(x, router, W1, W2)


# baseline (device time: 198145 ns/iter reference)
import jax
import jax.numpy as jnp
from jax import lax
from jax.experimental import pallas as pl
from jax.experimental.pallas import tpu as pltpu

N_DEV = 8


def kernel(x, router, W1, W2):
    t_per, d = x.shape
    e_per = W1.shape[0]
    f = W1.shape[2]
    t = t_per * N_DEV
    e_tot = e_per * N_DEV

    def body(x_ref, r_ref, w1_hbm, w2_hbm, o_ref,
             xf, rfT, wts, acc, w1buf, w2buf, commx, commr, commrs,
             w_sem, sx_send, sx_recv, sr_send, sr_recv, ss_send, ss_recv):
        e = pl.program_id(0)
        my = lax.axis_index("i")
        left = lax.rem(my - 1 + N_DEV, N_DEV)
        right = lax.rem(my + 1, N_DEV)

        w1_copy = pltpu.make_async_copy(w1_hbm.at[e], w1buf, w_sem.at[0])
        w2_copy = pltpu.make_async_copy(w2_hbm.at[e], w2buf, w_sem.at[1])
        w1_copy.start()
        w2_copy.start()

        @pl.when(e == 0)
        def _gather_and_route():
            barrier = pltpu.get_barrier_semaphore()
            for nbr in (left, right):
                pl.semaphore_signal(
                    barrier, inc=1, device_id=(nbr,),
                    device_id_type=pl.DeviceIdType.MESH,
                )
            pl.semaphore_wait(barrier, 2)

            xf[my] = x_ref[...]
            commx[0] = x_ref[...]
            for h in range(N_DEV - 1):
                s_slot = h % 2
                r_slot = (h + 1) % 2
                rdma = pltpu.make_async_remote_copy(
                    src_ref=commx.at[s_slot],
                    dst_ref=commx.at[r_slot],
                    send_sem=sx_send.at[s_slot],
                    recv_sem=sx_recv.at[r_slot],
                    device_id=(right,),
                    device_id_type=pl.DeviceIdType.MESH,
                )
                rdma.start()
                rdma.wait()
                origin = lax.rem(my - h - 1 + N_DEV, N_DEV)
                xf[origin] = commx[r_slot]

            rT = jnp.transpose(r_ref[...])
            rfT[my] = rT
            commr[0] = rT
            for h in range(N_DEV - 1):
                s_slot = h % 2
                r_slot = (h + 1) % 2
                rdma = pltpu.make_async_remote_copy(
                    src_ref=commr.at[s_slot],
                    dst_ref=commr.at[r_slot],
                    send_sem=sr_send.at[s_slot],
                    recv_sem=sr_recv.at[r_slot],
                    device_id=(right,),
                    device_id_type=pl.DeviceIdType.MESH,
                )
                rdma.start()
                rdma.wait()
                origin = lax.rem(my - h - 1 + N_DEV, N_DEV)
                rfT[origin] = commr[r_slot]

            rT_full = rfT[...].reshape(e_tot, d)
            for tc in range(N_DEV):
                g = lax.dot_general(
                    xf[tc], rT_full,
                    (((1,), (1,)), ((), ())),
                    precision=lax.Precision.HIGHEST,
                    preferred_element_type=jnp.float32,
                )
                wts[tc * t_per:(tc + 1) * t_per, :] = g

            gates = wts[...]
            m1 = jnp.max(gates, axis=1, keepdims=True)
            masked = jnp.where(gates >= m1, -jnp.inf, gates)
            m2 = jnp.max(masked, axis=1, keepdims=True)
            a = jnp.exp(m2 - m1)
            w_top = 1.0 / (1.0 + a)
            w_sec = a / (1.0 + a)
            wts[...] = jnp.where(gates >= m1, w_top, 0.0) + jnp.where(
                (gates >= m2) & (gates < m1), w_sec, 0.0
            )

        w1_copy.wait()
        w2_copy.wait()
        glob_e = my * e_per + e
        lane = lax.broadcasted_iota(jnp.int32, (t_per, e_tot), 1)
        for tc in range(N_DEV):
            col = jnp.sum(
                jnp.where(lane == glob_e,
                          wts[tc * t_per:(tc + 1) * t_per, :], 0.0),
                axis=1, keepdims=True,
            )
            h_act = jnp.maximum(
                jnp.dot(xf[tc], w1buf[...], preferred_element_type=jnp.float32),
                0.0,
            )
            y = jnp.dot(h_act, w2buf[...], preferred_element_type=jnp.float32)
            contrib = y * col

            @pl.when(e == 0)
            def _init_acc(tc=tc, contrib=contrib):
                acc[tc] = contrib

            @pl.when(e > 0)
            def _add_acc(tc=tc, contrib=contrib):
                acc[tc] = acc[tc] + contrib

        @pl.when(e == e_per - 1)
        def _reduce_scatter():
            c0 = lax.rem(my - 1 + N_DEV, N_DEV)
            commrs[0] = acc[c0]
            for s in range(N_DEV - 1):
                s_slot = s % 2
                r_slot = (s + 1) % 2
                rdma = pltpu.make_async_remote_copy(
                    src_ref=commrs.at[s_slot],
                    dst_ref=commrs.at[r_slot],
                    send_sem=ss_send.at[s_slot],
                    recv_sem=ss_recv.at[r_slot],
                    device_id=(right,),
                    device_id_type=pl.DeviceIdType.MESH,
                )
                rdma.start()
                rdma.wait()
                chunk = lax.rem(my - 2 - s + 2 * N_DEV, N_DEV)
                commrs[r_slot] = commrs[r_slot] + acc[chunk]
            o_ref[...] = commrs[(N_DEV - 1) % 2]

    grid = (e_per,)
    return pl.pallas_call(
        body,
        grid=grid,
        in_specs=[
            pl.BlockSpec((t_per, d), lambda e: (0, 0)),
            pl.BlockSpec((t, e_per), lambda e: (0, 0)),
            pl.BlockSpec(memory_space=pltpu.MemorySpace.HBM),
            pl.BlockSpec(memory_space=pltpu.MemorySpace.HBM),
        ],
        out_specs=pl.BlockSpec((t_per, d), lambda e: (0, 0)),
        out_shape=jax.ShapeDtypeStruct((t_per, d), jnp.float32),
        scratch_shapes=[
            pltpu.VMEM((N_DEV, t_per, d), jnp.float32),
            pltpu.VMEM((N_DEV, e_per, d), jnp.float32),
            pltpu.VMEM((t, e_tot), jnp.float32),
            pltpu.VMEM((N_DEV, t_per, d), jnp.float32),
            pltpu.VMEM((d, f), jnp.float32),
            pltpu.VMEM((f, d), jnp.float32),
            pltpu.VMEM((2, t_per, d), jnp.float32),
            pltpu.VMEM((2, e_per, d), jnp.float32),
            pltpu.VMEM((2, t_per, d), jnp.float32),
            pltpu.SemaphoreType.DMA((2,)),
            pltpu.SemaphoreType.DMA((2,)),
            pltpu.SemaphoreType.DMA((2,)),
            pltpu.SemaphoreType.DMA((2,)),
            pltpu.SemaphoreType.DMA((2,)),
            pltpu.SemaphoreType.DMA((2,)),
            pltpu.SemaphoreType.DMA((2,)),
        ],
        compiler_params=pltpu.CompilerParams(
            dimension_semantics=("arbitrary",),
            collective_id=0,
            vmem_limit_bytes=60 * 1024 * 1024,
        ),
    )(x, router, W1, W2)


# device time: 105892 ns/iter; 1.8712x vs baseline; 1.8712x over previous
import jax
import jax.numpy as jnp
from jax import lax
from jax.experimental import pallas as pl
from jax.experimental.pallas import tpu as pltpu

N_DEV = 8


def kernel(x, router, W1, W2):
    t_per, d = x.shape
    e_per = W1.shape[0]
    f = W1.shape[2]
    t = t_per * N_DEV
    e_tot = e_per * N_DEV

    def body(x_ref, r_ref, w1_hbm, w2_hbm, o_ref,
             xf, rfT, rstage, wts, acc, rsbuf, w1buf, w2buf,
             w_sems, xa_send, xa_recv, ra_send, ra_recv, rs_send, rs_recv):
        e = pl.program_id(0)
        my = lax.axis_index("i")

        def peer(j):
            return lax.rem(my + j, N_DEV)

        def w_copies(k, parity):
            return (
                pltpu.make_async_copy(
                    w1_hbm.at[k], w1buf.at[parity], w_sems.at[parity, 0]),
                pltpu.make_async_copy(
                    w2_hbm.at[k], w2buf.at[parity], w_sems.at[parity, 1]),
            )

        @pl.when(e == 0)
        def _fetch_first():
            for c in w_copies(0, 0):
                c.start()

        @pl.when(e < e_per - 1)
        def _prefetch_next():
            for c in w_copies(e + 1, (e + 1) % 2):
                c.start()

        def wait_w(parity):
            for c in w_copies(0, parity):
                c.wait()

        def ffn(xc, col):
            h_act = jnp.maximum(
                jnp.dot(xc, w1buf[e % 2], preferred_element_type=jnp.float32),
                0.0,
            )
            y = jnp.dot(h_act, w2buf[e % 2],
                        preferred_element_type=jnp.float32)
            return y * col

        lane = lax.broadcasted_iota(jnp.int32, (t_per, e_tot), 1)
        glob_e = my * e_per + e

        def col_for(rows):
            return jnp.sum(
                jnp.where(lane == glob_e, rows, 0.0), axis=1, keepdims=True)

        @pl.when(e == 0)
        def _gather_and_route():
            barrier = pltpu.get_barrier_semaphore()
            for j in range(1, N_DEV):
                pl.semaphore_signal(
                    barrier, inc=1, device_id=(peer(j),),
                    device_id_type=pl.DeviceIdType.MESH,
                )
            pl.semaphore_wait(barrier, N_DEV - 1)

            x_sends = []
            for j in range(1, N_DEV):
                rdma = pltpu.make_async_remote_copy(
                    src_ref=x_ref,
                    dst_ref=xf.at[my],
                    send_sem=xa_send.at[peer(j)],
                    recv_sem=xa_recv.at[my],
                    device_id=(peer(j),),
                    device_id_type=pl.DeviceIdType.MESH,
                )
                rdma.start()
                x_sends.append(rdma)
            xf[my] = x_ref[...]

            rstage[...] = jnp.transpose(r_ref[...])
            rfT[my] = rstage[...]
            r_sends = []
            for j in range(1, N_DEV):
                rdma = pltpu.make_async_remote_copy(
                    src_ref=rstage,
                    dst_ref=rfT.at[my],
                    send_sem=ra_send.at[peer(j)],
                    recv_sem=ra_recv.at[my],
                    device_id=(peer(j),),
                    device_id_type=pl.DeviceIdType.MESH,
                )
                rdma.start()
                r_sends.append(rdma)

            wait_w(0)
            acc[my] = ffn(xf[my], 1.0)
            for j in range(1, N_DEV):
                p = peer(j)
                pltpu.make_async_remote_copy(
                    src_ref=x_ref, dst_ref=xf.at[p],
                    send_sem=xa_send.at[p], recv_sem=xa_recv.at[p],
                    device_id=(p,), device_id_type=pl.DeviceIdType.MESH,
                ).wait_recv()
                acc[p] = ffn(xf[p], 1.0)

            for j in range(1, N_DEV):
                p = peer(j)
                pltpu.make_async_remote_copy(
                    src_ref=rstage, dst_ref=rfT.at[p],
                    send_sem=ra_send.at[p], recv_sem=ra_recv.at[p],
                    device_id=(p,), device_id_type=pl.DeviceIdType.MESH,
                ).wait_recv()

            rT_full = rfT[...].reshape(e_tot, d)
            for tc in range(N_DEV):
                g = lax.dot_general(
                    xf[tc], rT_full,
                    (((1,), (1,)), ((), ())),
                    precision=lax.Precision.HIGHEST,
                    preferred_element_type=jnp.float32,
                )
                wts[tc * t_per:(tc + 1) * t_per, :] = g

            gates = wts[...]
            m1 = jnp.max(gates, axis=1, keepdims=True)
            masked = jnp.where(gates >= m1, -jnp.inf, gates)
            m2 = jnp.max(masked, axis=1, keepdims=True)
            a = jnp.exp(m2 - m1)
            w_top = 1.0 / (1.0 + a)
            w_sec = a / (1.0 + a)
            wts[...] = jnp.where(gates >= m1, w_top, 0.0) + jnp.where(
                (gates >= m2) & (gates < m1), w_sec, 0.0
            )

            for tc in range(N_DEV):
                acc[tc] = acc[tc] * col_for(
                    wts[tc * t_per:(tc + 1) * t_per, :])

            for rdma in x_sends + r_sends:
                rdma.wait_send()

        @pl.when((e > 0) & (e < e_per - 1))
        def _mid_experts():
            wait_w(e % 2)
            for tc in range(N_DEV):
                col = col_for(wts[tc * t_per:(tc + 1) * t_per, :])
                acc[tc] = acc[tc] + ffn(xf[tc], col)

        @pl.when(e == e_per - 1)
        def _last_expert_and_reduce_scatter():
            wait_w((e_per - 1) % 2)
            rs_sends = []
            for j in range(1, N_DEV):
                c = peer(j)
                col = col_for(wts[pl.ds(c * t_per, t_per), :])
                acc[c] = acc[c] + ffn(xf[c], col)
                rdma = pltpu.make_async_remote_copy(
                    src_ref=acc.at[c],
                    dst_ref=rsbuf.at[my],
                    send_sem=rs_send.at[c],
                    recv_sem=rs_recv.at[my],
                    device_id=(c,),
                    device_id_type=pl.DeviceIdType.MESH,
                )
                rdma.start()
                rs_sends.append(rdma)
            col = col_for(wts[pl.ds(my * t_per, t_per), :])
            total = acc[my] + ffn(xf[my], col)
            for j in range(1, N_DEV):
                p = peer(j)
                pltpu.make_async_remote_copy(
                    src_ref=acc.at[p], dst_ref=rsbuf.at[p],
                    send_sem=rs_send.at[p], recv_sem=rs_recv.at[p],
                    device_id=(p,), device_id_type=pl.DeviceIdType.MESH,
                ).wait_recv()
                total = total + rsbuf[p]
            o_ref[...] = total
            for rdma in rs_sends:
                rdma.wait_send()

    grid = (e_per,)
    return pl.pallas_call(
        body,
        grid=grid,
        in_specs=[
            pl.BlockSpec((t_per, d), lambda e: (0, 0)),
            pl.BlockSpec((t, e_per), lambda e: (0, 0)),
            pl.BlockSpec(memory_space=pltpu.MemorySpace.HBM),
            pl.BlockSpec(memory_space=pltpu.MemorySpace.HBM),
        ],
        out_specs=pl.BlockSpec((t_per, d), lambda e: (0, 0)),
        out_shape=jax.ShapeDtypeStruct((t_per, d), jnp.float32),
        scratch_shapes=[
            pltpu.VMEM((N_DEV, t_per, d), jnp.float32),
            pltpu.VMEM((N_DEV, e_per, d), jnp.float32),
            pltpu.VMEM((e_per, d), jnp.float32),
            pltpu.VMEM((t, e_tot), jnp.float32),
            pltpu.VMEM((N_DEV, t_per, d), jnp.float32),
            pltpu.VMEM((N_DEV, t_per, d), jnp.float32),
            pltpu.VMEM((2, d, f), jnp.float32),
            pltpu.VMEM((2, f, d), jnp.float32),
            pltpu.SemaphoreType.DMA((2, 2)),
            pltpu.SemaphoreType.DMA((N_DEV,)),
            pltpu.SemaphoreType.DMA((N_DEV,)),
            pltpu.SemaphoreType.DMA((N_DEV,)),
            pltpu.SemaphoreType.DMA((N_DEV,)),
            pltpu.SemaphoreType.DMA((N_DEV,)),
            pltpu.SemaphoreType.DMA((N_DEV,)),
        ],
        compiler_params=pltpu.CompilerParams(
            dimension_semantics=("arbitrary",),
            collective_id=0,
            vmem_limit_bytes=60 * 1024 * 1024,
        ),
    )(x, router, W1, W2)


# device time: 80288 ns/iter; 2.4679x vs baseline; 1.3189x over previous
import jax
import jax.numpy as jnp
from jax import lax
from jax.experimental import pallas as pl
from jax.experimental.pallas import tpu as pltpu

N_DEV = 8


def kernel(x, router, W1, W2):
    t_per, d = x.shape
    e_per = W1.shape[0]
    f = W1.shape[2]
    e_tot = e_per * N_DEV

    def body(x_ref, r_ref, w1_hbm, w2_hbm, o_ref,
             xbf, xf, rfT, rstage, wtsb, acc, accb, rsbufb, w1buf, w2buf,
             w_sems, xa_send, xa_recv, ra_send, ra_recv,
             wb_send, wb_recv, rs_send, rs_recv):
        e = pl.program_id(0)
        my = lax.axis_index("i")

        def peer(j):
            return lax.rem(my + j, N_DEV)

        def w_copies(k, parity):
            return (
                pltpu.make_async_copy(
                    w1_hbm.at[k], w1buf.at[parity], w_sems.at[parity, 0]),
                pltpu.make_async_copy(
                    w2_hbm.at[k], w2buf.at[parity], w_sems.at[parity, 1]),
            )

        @pl.when(e == 0)
        def _fetch_first():
            for c in w_copies(0, 0):
                c.start()

        @pl.when(e < e_per - 1)
        def _prefetch_next():
            for c in w_copies(e + 1, (e + 1) % 2):
                c.start()

        def wait_w(parity):
            for c in w_copies(0, parity):
                c.wait()

        lane = lax.broadcasted_iota(jnp.int32, (t_per, e_tot), 1)
        glob_e = my * e_per + e

        def col_for(rows):
            return jnp.sum(
                jnp.where(lane == glob_e, rows, 0.0), axis=1, keepdims=True)

        def make_ffn():
            w1b = w1buf[e % 2].astype(jnp.bfloat16)
            w2b = w2buf[e % 2].astype(jnp.bfloat16)

            def ffn(xc, col):
                h_act = jnp.maximum(
                    jnp.dot(xc, w1b, preferred_element_type=jnp.float32),
                    0.0,
                )
                y = jnp.dot(h_act.astype(jnp.bfloat16), w2b,
                            preferred_element_type=jnp.float32)
                return y * col
            return ffn

        @pl.when(e == 0)
        def _gather_and_route():
            barrier = pltpu.get_barrier_semaphore()
            for j in range(1, N_DEV):
                pl.semaphore_signal(
                    barrier, inc=1, device_id=(peer(j),),
                    device_id_type=pl.DeviceIdType.MESH,
                )
            pl.semaphore_wait(barrier, N_DEV - 1)

            xbf[...] = x_ref[...].astype(jnp.bfloat16)
            sends = []
            for j in range(1, N_DEV):
                rdma = pltpu.make_async_remote_copy(
                    src_ref=xbf,
                    dst_ref=xf.at[my],
                    send_sem=xa_send.at[peer(j)],
                    recv_sem=xa_recv.at[my],
                    device_id=(peer(j),),
                    device_id_type=pl.DeviceIdType.MESH,
                )
                rdma.start()
                sends.append(rdma)
            xf[my] = xbf[...]

            rstage[...] = jnp.transpose(r_ref[...])
            rfT[my] = rstage[...]
            for j in range(1, N_DEV):
                rdma = pltpu.make_async_remote_copy(
                    src_ref=rstage,
                    dst_ref=rfT.at[my],
                    send_sem=ra_send.at[peer(j)],
                    recv_sem=ra_recv.at[my],
                    device_id=(peer(j),),
                    device_id_type=pl.DeviceIdType.MESH,
                )
                rdma.start()
                sends.append(rdma)

            for j in range(1, N_DEV):
                p = peer(j)
                pltpu.make_async_remote_copy(
                    src_ref=rstage, dst_ref=rfT.at[p],
                    send_sem=ra_send.at[p], recv_sem=ra_recv.at[p],
                    device_id=(p,), device_id_type=pl.DeviceIdType.MESH,
                ).wait_recv()

            rT_full = rfT[...].reshape(e_tot, d)
            g = lax.dot_general(
                x_ref[...], rT_full,
                (((1,), (1,)), ((), ())),
                precision=lax.Precision.HIGHEST,
                preferred_element_type=jnp.float32,
            )

            m1 = jnp.max(g, axis=1, keepdims=True)
            masked = jnp.where(g >= m1, -jnp.inf, g)
            m2 = jnp.max(masked, axis=1, keepdims=True)
            a = jnp.exp(m2 - m1)
            w_top = 1.0 / (1.0 + a)
            w_sec = a / (1.0 + a)
            wtsb[my] = jnp.where(g >= m1, w_top, 0.0) + jnp.where(
                (g >= m2) & (g < m1), w_sec, 0.0
            )

            for j in range(1, N_DEV):
                rdma = pltpu.make_async_remote_copy(
                    src_ref=wtsb.at[my],
                    dst_ref=wtsb.at[my],
                    send_sem=wb_send.at[peer(j)],
                    recv_sem=wb_recv.at[my],
                    device_id=(peer(j),),
                    device_id_type=pl.DeviceIdType.MESH,
                )
                rdma.start()
                sends.append(rdma)

            wait_w(0)
            ffn = make_ffn()
            acc[my] = ffn(xf[my], col_for(wtsb[my]))
            for j in range(1, N_DEV):
                p = peer(j)
                pltpu.make_async_remote_copy(
                    src_ref=xbf, dst_ref=xf.at[p],
                    send_sem=xa_send.at[p], recv_sem=xa_recv.at[p],
                    device_id=(p,), device_id_type=pl.DeviceIdType.MESH,
                ).wait_recv()
                pltpu.make_async_remote_copy(
                    src_ref=wtsb.at[p], dst_ref=wtsb.at[p],
                    send_sem=wb_send.at[p], recv_sem=wb_recv.at[p],
                    device_id=(p,), device_id_type=pl.DeviceIdType.MESH,
                ).wait_recv()
                acc[p] = ffn(xf[p], col_for(wtsb[p]))

            for rdma in sends:
                rdma.wait_send()

        @pl.when((e > 0) & (e < e_per - 1))
        def _mid_experts():
            wait_w(e % 2)
            ffn = make_ffn()
            for tc in range(N_DEV):
                acc[tc] = acc[tc] + ffn(xf[tc], col_for(wtsb[tc]))

        @pl.when(e == e_per - 1)
        def _last_expert_and_reduce_scatter():
            wait_w((e_per - 1) % 2)
            ffn = make_ffn()
            rs_sends = []
            for j in range(1, N_DEV):
                c = peer(j)
                accb[c] = (acc[c] + ffn(xf[c], col_for(wtsb[c]))).astype(
                    jnp.bfloat16)
                rdma = pltpu.make_async_remote_copy(
                    src_ref=accb.at[c],
                    dst_ref=rsbufb.at[my],
                    send_sem=rs_send.at[c],
                    recv_sem=rs_recv.at[my],
                    device_id=(c,),
                    device_id_type=pl.DeviceIdType.MESH,
                )
                rdma.start()
                rs_sends.append(rdma)
            total = acc[my] + ffn(xf[my], col_for(wtsb[my]))
            for j in range(1, N_DEV):
                p = peer(j)
                pltpu.make_async_remote_copy(
                    src_ref=accb.at[p], dst_ref=rsbufb.at[p],
                    send_sem=rs_send.at[p], recv_sem=rs_recv.at[p],
                    device_id=(p,), device_id_type=pl.DeviceIdType.MESH,
                ).wait_recv()
                total = total + rsbufb[p].astype(jnp.float32)
            o_ref[...] = total
            for rdma in rs_sends:
                rdma.wait_send()

    grid = (e_per,)
    return pl.pallas_call(
        body,
        grid=grid,
        in_specs=[
            pl.BlockSpec((t_per, d), lambda e: (0, 0)),
            pl.BlockSpec((t_per * N_DEV, e_per), lambda e: (0, 0)),
            pl.BlockSpec(memory_space=pltpu.MemorySpace.HBM),
            pl.BlockSpec(memory_space=pltpu.MemorySpace.HBM),
        ],
        out_specs=pl.BlockSpec((t_per, d), lambda e: (0, 0)),
        out_shape=jax.ShapeDtypeStruct((t_per, d), jnp.float32),
        scratch_shapes=[
            pltpu.VMEM((t_per, d), jnp.bfloat16),
            pltpu.VMEM((N_DEV, t_per, d), jnp.bfloat16),
            pltpu.VMEM((N_DEV, e_per, d), jnp.float32),
            pltpu.VMEM((e_per, d), jnp.float32),
            pltpu.VMEM((N_DEV, t_per, e_tot), jnp.float32),
            pltpu.VMEM((N_DEV, t_per, d), jnp.float32),
            pltpu.VMEM((N_DEV, t_per, d), jnp.bfloat16),
            pltpu.VMEM((N_DEV, t_per, d), jnp.bfloat16),
            pltpu.VMEM((2, d, f), jnp.float32),
            pltpu.VMEM((2, f, d), jnp.float32),
            pltpu.SemaphoreType.DMA((2, 2)),
            pltpu.SemaphoreType.DMA((N_DEV,)),
            pltpu.SemaphoreType.DMA((N_DEV,)),
            pltpu.SemaphoreType.DMA((N_DEV,)),
            pltpu.SemaphoreType.DMA((N_DEV,)),
            pltpu.SemaphoreType.DMA((N_DEV,)),
            pltpu.SemaphoreType.DMA((N_DEV,)),
            pltpu.SemaphoreType.DMA((N_DEV,)),
            pltpu.SemaphoreType.DMA((N_DEV,)),
        ],
        compiler_params=pltpu.CompilerParams(
            dimension_semantics=("arbitrary",),
            collective_id=0,
            vmem_limit_bytes=60 * 1024 * 1024,
        ),
    )(x, router, W1, W2)
